# baseline (speedup 1.0000x reference)
import functools

import jax
import jax.numpy as jnp
from jax import lax
from jax.experimental import pallas as pl
from jax.experimental.pallas import tpu as pltpu
from jax.experimental.pallas import tpu_sc as plsc

B = 16384
D = 64
L = 16
NW = 32
BW = B // NW
CHUNK = 128
NCH = BW // CHUNK


@functools.partial(
    pl.kernel,
    out_type=[
        jax.ShapeDtypeStruct((B,), jnp.float32),
        jax.ShapeDtypeStruct((B,), jnp.float32),
        jax.ShapeDtypeStruct((B,), jnp.float32),
    ],
    mesh=plsc.VectorSubcoreMesh(core_axis_name="c", subcore_axis_name="s"),
    compiler_params=pltpu.CompilerParams(
        needs_layout_passes=False, use_tc_tiling_on_sc=False),
    scratch_types=[
        pltpu.VMEM((NCH, CHUNK), jnp.int32),
        pltpu.VMEM((NCH, CHUNK), jnp.int32),
        pltpu.VMEM((NCH, CHUNK), jnp.int32),
        pltpu.VMEM((BW, D), jnp.float32),
        pltpu.VMEM((BW, D), jnp.float32),
        pltpu.VMEM((BW, D), jnp.float32),
        pltpu.VMEM((BW,), jnp.float32),
        pltpu.VMEM((BW,), jnp.float32),
        pltpu.VMEM((BW,), jnp.float32),
        pltpu.VMEM((L * L,), jnp.float32),
        pltpu.VMEM((L * L,), jnp.float32),
        pltpu.SemaphoreType.DMA,
    ],
)
def _ubpr_sc(bu_hbm, bi_hbm, bj_hbm, ue_hbm, ie_hbm, prop_hbm,
             pos_hbm, neg_hbm, ppos_hbm,
             idx_u, idx_i, idx_j, u_v, i_v, j_v, prop_v, pos_v, neg_v,
             tr_p, tr_n, sem):
    wid = lax.axis_index("s") * 2 + lax.axis_index("c")
    base = wid * BW

    row0 = wid * NCH
    pltpu.sync_copy(bu_hbm.at[pl.ds(row0, NCH)], idx_u)
    pltpu.sync_copy(bi_hbm.at[pl.ds(row0, NCH)], idx_i)
    pltpu.sync_copy(bj_hbm.at[pl.ds(row0, NCH)], idx_j)

    copies = []
    for c in range(NCH):
        dst = pl.ds(c * CHUNK, CHUNK)
        copies.append(pltpu.async_copy(ue_hbm.at[idx_u.at[c]], u_v.at[dst], sem))
        copies.append(pltpu.async_copy(ie_hbm.at[idx_i.at[c]], i_v.at[dst], sem))
        copies.append(pltpu.async_copy(ie_hbm.at[idx_j.at[c]], j_v.at[dst], sem))
        copies.append(pltpu.async_copy(prop_hbm.at[idx_i.at[c]], prop_v.at[dst], sem))
    for cp in copies:
        cp.wait()

    lane = lax.iota(jnp.int32, L)

    def body(g, _):
        for el in range(L):
            e = g * L + el
            acc_p = None
            acc_n = None
            for c in range(D // L):
                sl = pl.ds(c * L, L)
                uu = u_v[e, sl]
                pp = uu * i_v[e, sl]
                nn = uu * j_v[e, sl]
                acc_p = pp if acc_p is None else acc_p + pp
                acc_n = nn if acc_n is None else acc_n + nn
            col = lane * L + el
            plsc.store_scatter(tr_p, [col], acc_p)
            plsc.store_scatter(tr_n, [col], acc_n)
        sum_p = None
        sum_n = None
        for r in range(L):
            rsl = pl.ds(r * L, L)
            rp = tr_p[rsl]
            rn = tr_n[rsl]
            sum_p = rp if sum_p is None else sum_p + rp
            sum_n = rn if sum_n is None else sum_n + rn
        out_sl = pl.ds(g * L, L)
        pos_v[out_sl] = sum_p
        neg_v[out_sl] = sum_n
        return 0

    lax.fori_loop(0, BW // L, body, 0, unroll=False)

    def clamp(g, _):
        sl = pl.ds(g * L, L)
        prop_v[sl] = jnp.maximum(prop_v[sl], 0.1)
        return 0

    lax.fori_loop(0, BW // L, clamp, 0, unroll=False)

    out = pl.ds(base, BW)
    pltpu.sync_copy(pos_v, pos_hbm.at[out])
    pltpu.sync_copy(neg_v, neg_hbm.at[out])
    pltpu.sync_copy(prop_v, ppos_hbm.at[out])


@jax.jit
def kernel(batch_user, batch_pos_item, batch_neg_item, user_emb, item_emb,
           i_propensity):
    bu = batch_user.astype(jnp.int32).reshape(NW * NCH, CHUNK)
    bi = batch_pos_item.astype(jnp.int32).reshape(NW * NCH, CHUNK)
    bj = batch_neg_item.astype(jnp.int32).reshape(NW * NCH, CHUNK)
    pos, neg, ppos = _ubpr_sc(bu, bi, bj, user_emb, item_emb, i_propensity)
    return pos.reshape(B, 1), neg.reshape(B, 1), ppos


# speedup vs baseline: 1.0303x; 1.0303x over previous
import functools

import jax
import jax.numpy as jnp
from jax import lax
from jax.experimental import pallas as pl
from jax.experimental.pallas import tpu as pltpu
from jax.experimental.pallas import tpu_sc as plsc

B = 16384
D = 64
DP = 128
L = 16
NW = 32
BW = B // NW
CHUNK = 128
NCH = BW // CHUNK


@functools.partial(
    pl.kernel,
    out_type=[
        jax.ShapeDtypeStruct((B,), jnp.float32),
        jax.ShapeDtypeStruct((B,), jnp.float32),
        jax.ShapeDtypeStruct((B,), jnp.float32),
    ],
    mesh=plsc.VectorSubcoreMesh(core_axis_name="c", subcore_axis_name="s"),
    compiler_params=pltpu.CompilerParams(
        needs_layout_passes=False, use_tc_tiling_on_sc=False),
    scratch_types=[
        pltpu.VMEM((NCH, CHUNK), jnp.int32),
        pltpu.VMEM((NCH, CHUNK), jnp.int32),
        pltpu.VMEM((NCH, CHUNK), jnp.int32),
        pltpu.VMEM((2, CHUNK, DP), jnp.float32),
        pltpu.VMEM((2, CHUNK, DP), jnp.float32),
        pltpu.VMEM((2, CHUNK, DP), jnp.float32),
        pltpu.VMEM((BW,), jnp.float32),
        pltpu.VMEM((BW,), jnp.float32),
        pltpu.VMEM((BW,), jnp.float32),
        pltpu.SemaphoreType.DMA,
        pltpu.SemaphoreType.DMA,
    ],
)
def _ubpr_sc(bu_hbm, bi_hbm, bj_hbm, ue_hbm, ie_hbm, prop_hbm,
             pos_hbm, neg_hbm, ppos_hbm,
             idx_u, idx_i, idx_j, u_v, i_v, j_v, prop_v, pos_v, neg_v,
             sem0, sem1):
    wid = lax.axis_index("s") * 2 + lax.axis_index("c")
    base = wid * BW
    sems = (sem0, sem1)

    for c in range(NCH):
        src = pl.ds(base + c * CHUNK, CHUNK)
        pltpu.sync_copy(bu_hbm.at[src], idx_u.at[c])
        pltpu.sync_copy(bi_hbm.at[src], idx_i.at[c])
        pltpu.sync_copy(bj_hbm.at[src], idx_j.at[c])

    def fire(c):
        buf = c % 2
        sem = sems[buf]
        return [
            pltpu.async_copy(ue_hbm.at[idx_u.at[c]], u_v.at[buf], sem),
            pltpu.async_copy(ie_hbm.at[idx_i.at[c]], i_v.at[buf], sem),
            pltpu.async_copy(ie_hbm.at[idx_j.at[c]], j_v.at[buf], sem),
            pltpu.async_copy(prop_hbm.at[idx_i.at[c]],
                             prop_v.at[pl.ds(c * CHUNK, CHUNK)], sem),
        ]

    lane = lax.iota(jnp.int32, L)
    last = lane == (L - 1)
    UNROLL = 16

    pending = fire(0)
    for c in range(NCH):
        for cp in pending:
            cp.wait()
        if c + 1 < NCH:
            pending = fire(c + 1)
        buf = c % 2
        ub = u_v.at[buf]
        ib = i_v.at[buf]
        jb = j_v.at[buf]

        def body(g, _):
            for el in range(UNROLL):
                e = g * UNROLL + el
                acc_p = None
                acc_n = None
                for k in range(D // L):
                    sl = pl.ds(k * L, L)
                    uu = ub[e, sl]
                    pp = uu * ib[e, sl]
                    nn = uu * jb[e, sl]
                    acc_p = pp if acc_p is None else acc_p + pp
                    acc_n = nn if acc_n is None else acc_n + nn
                eidx = jnp.full((L,), c * CHUNK + e, jnp.int32)
                plsc.store_scatter(pos_v, [eidx], plsc.cumsum(acc_p), mask=last)
                plsc.store_scatter(neg_v, [eidx], plsc.cumsum(acc_n), mask=last)
            return 0

        lax.fori_loop(0, CHUNK // UNROLL, body, 0, unroll=False)

    def clamp(g, _):
        sl = pl.ds(g * L, L)
        prop_v[sl] = jnp.maximum(prop_v[sl], 0.1)
        return 0

    lax.fori_loop(0, BW // L, clamp, 0, unroll=False)

    out = pl.ds(base, BW)
    pltpu.sync_copy(pos_v, pos_hbm.at[out])
    pltpu.sync_copy(neg_v, neg_hbm.at[out])
    pltpu.sync_copy(prop_v, ppos_hbm.at[out])


@jax.jit
def kernel(batch_user, batch_pos_item, batch_neg_item, user_emb, item_emb,
           i_propensity):
    bu = batch_user.astype(jnp.int32)
    bi = batch_pos_item.astype(jnp.int32)
    bj = batch_neg_item.astype(jnp.int32)
    uep = jnp.pad(user_emb, ((0, 0), (0, DP - D)))
    iep = jnp.pad(item_emb, ((0, 0), (0, DP - D)))
    pos, neg, ppos = _ubpr_sc(bu, bi, bj, uep, iep, i_propensity)
    return pos.reshape(B, 1), neg.reshape(B, 1), ppos
